# Initial kernel scaffold; baseline (speedup 1.0000x reference)
import jax
import jax.numpy as jnp
from jax.experimental import pallas as pl


def kernel(features):
    raise NotImplementedError("write your pallas kernel here")


# speedup vs baseline: 5.6094x; 5.6094x over previous
import functools

import jax
import jax.numpy as jnp
from jax.experimental import pallas as pl
from jax.experimental.pallas import tpu as pltpu

_K = 16


def _body(f_all_ref, f_rows_ref, adj_ref, *, block_rows, n):
    i = pl.program_id(0)
    f_rows = f_rows_ref[...]
    f_all = f_all_ref[...]
    rn = jnp.sum(f_rows * f_rows, axis=1, keepdims=True)
    cn = jnp.sum(f_all * f_all, axis=1)[None, :]
    prod = jax.lax.dot_general(
        f_rows, f_all, (((1,), (1,)), ((), ())),
        preferred_element_type=jnp.float32)
    dist = rn + cn - 2.0 * prod
    sim = jnp.exp(-jnp.maximum(dist, 0.0))
    col = jax.lax.broadcasted_iota(jnp.int32, (block_rows, n), 1)
    row_g = i * block_rows + jax.lax.broadcasted_iota(
        jnp.int32, (block_rows, n), 0)
    sim = jnp.where(col == row_g, 0.0, sim)

    adj = jnp.zeros((block_rows, n), jnp.float32)
    curr = sim
    for _ in range(_K):
        m = jnp.max(curr, axis=1, keepdims=True)
        cand = jnp.where(curr == m, col, n)
        amin = jnp.min(cand, axis=1, keepdims=True)
        hit = col == amin
        adj = jnp.where(hit, 1.0, adj)
        curr = jnp.where(hit, -1.0, curr)
    adj_ref[...] = adj


@functools.partial(jax.jit, static_argnames=("block_rows",))
def _run(features, block_rows=256):
    n, d = features.shape
    grid = n // block_rows
    return pl.pallas_call(
        functools.partial(_body, block_rows=block_rows, n=n),
        grid=(grid,),
        in_specs=[
            pl.BlockSpec((n, d), lambda i: (0, 0)),
            pl.BlockSpec((block_rows, d), lambda i: (i, 0)),
        ],
        out_specs=pl.BlockSpec((block_rows, n), lambda i: (i, 0)),
        out_shape=jax.ShapeDtypeStruct((n, n), jnp.float32),
        compiler_params=pltpu.CompilerParams(
            dimension_semantics=("arbitrary",),
        ),
    )(features, features)


def kernel(features):
    return _run(features)
